# baseline (speedup 1.0000x reference)
import functools

import jax
import jax.numpy as jnp
from jax import lax
from jax.experimental import pallas as pl
from jax.experimental.pallas import tpu as pltpu
from jax.experimental.pallas import tpu_sc as plsc

N = 10000
E = 320000
D = 128
OUT = 16
L = 4
BN_EPS = 1e-5

NC = 2
NS = 16
NW = NC * NS

CHUNK = 64
E_PAD = 327680
ECH = E_PAD // CHUNK
N_CHUNKS = ECH // NW
N_AGG = 10016

RCHUNK = 16
NRC = N // RCHUNK
NRZ = N_AGG // RCHUNK


def _seg_sum_body(src_hbm, dst_hbm, h_hbm, out_hbm,
                  src_v, dst_v, rows_a, rows_b, zero_v,
                  agg_sh, sem_a, sem_b):
    c = lax.axis_index("c")
    s = lax.axis_index("s")
    wid = c * NS + s

    cp_src = pltpu.async_copy(
        src_hbm.at[pl.ds(wid * N_CHUNKS * CHUNK, N_CHUNKS * CHUNK)],
        src_v, sem_a)
    cp_dst = pltpu.async_copy(dst_hbm.at[wid], dst_v, sem_b)

    @pl.loop(0, RCHUNK * D // 16)
    def _zero_vmem(i):
        r = i // (D // 16)
        k = i % (D // 16)
        zero_v[r, pl.ds(k * 16, 16)] = jnp.zeros((16,), jnp.float32)

    @pl.loop(0, (NRZ + NS - 1) // NS)
    def _zero_sh(j):
        ch = s + j * NS
        @pl.when(ch < NRZ)
        def _():
            pltpu.sync_copy(zero_v, agg_sh.at[pl.ds(ch * RCHUNK, RCHUNK)])

    cp_src.wait()
    cp_dst.wait()
    plsc.subcore_barrier()

    def _src_idx(j):
        return src_v.at[pl.ds(j * CHUNK, CHUNK)]

    pltpu.async_copy(h_hbm.at[_src_idx(0)], rows_a, sem_a)

    @pl.loop(0, N_CHUNKS)
    def _edges(j):
        even = j % 2 == 0

        @pl.when(even)
        def _():
            pltpu.make_async_copy(h_hbm.at[_src_idx(j)], rows_a, sem_a).wait()
            @pl.when(j + 1 < N_CHUNKS)
            def _():
                pltpu.async_copy(h_hbm.at[_src_idx(j + 1)], rows_b, sem_b)
            pltpu.sync_copy(rows_a, agg_sh.at[dst_v.at[j]], add=True)

        @pl.when(jnp.logical_not(even))
        def _():
            pltpu.make_async_copy(h_hbm.at[_src_idx(j)], rows_b, sem_b).wait()
            @pl.when(j + 1 < N_CHUNKS)
            def _():
                pltpu.async_copy(h_hbm.at[_src_idx(j + 1)], rows_a, sem_a)
            pltpu.sync_copy(rows_b, agg_sh.at[dst_v.at[j]], add=True)

    plsc.subcore_barrier()

    @pl.loop(0, (NRC + NS - 1) // NS)
    def _copy_out(j):
        ch = s + j * NS
        @pl.when(ch < NRC)
        def _():
            pltpu.sync_copy(agg_sh.at[pl.ds(ch * RCHUNK, RCHUNK)],
                            out_hbm.at[c].at[pl.ds(ch * RCHUNK, RCHUNK)])


@jax.jit
def _seg_sum(h, src, dst):
    mesh = plsc.VectorSubcoreMesh(core_axis_name="c", subcore_axis_name="s",
                                  num_cores=NC, num_subcores=NS)
    return pl.kernel(
        _seg_sum_body,
        out_type=jax.ShapeDtypeStruct((NC, N, D), jnp.float32),
        mesh=mesh,
        scratch_types=[
            pltpu.VMEM((N_CHUNKS * CHUNK,), jnp.int32),
            pltpu.VMEM((N_CHUNKS, CHUNK), jnp.int32),
            pltpu.VMEM((CHUNK, D), jnp.float32),
            pltpu.VMEM((CHUNK, D), jnp.float32),
            pltpu.VMEM((RCHUNK, D), jnp.float32),
            pltpu.VMEM_SHARED((N_AGG, D), jnp.float32),
            pltpu.SemaphoreType.DMA,
            pltpu.SemaphoreType.DMA,
        ],
    )(src, dst, h)


def _dense_body(h_ref, p0_ref, p1_ref, w1_ref, g1_ref, b1_ref,
                w2_ref, g2_ref, b2_ref, h_out_ref, pooled_ref):
    r = h_ref[...] + p0_ref[...] + p1_ref[...]
    y = lax.dot_general(r, w1_ref[...], (((1,), (1,)), ((), ())),
                        preferred_element_type=jnp.float32)
    mean = jnp.mean(y, axis=0, keepdims=True)
    var = jnp.mean(jnp.square(y - mean), axis=0, keepdims=True)
    y = (y - mean) * lax.rsqrt(var + BN_EPS) * g1_ref[...] + b1_ref[...]
    y = jnp.maximum(y, 0.0)
    z = lax.dot_general(y, w2_ref[...], (((1,), (1,)), ((), ())),
                        preferred_element_type=jnp.float32)
    mean2 = jnp.mean(z, axis=0, keepdims=True)
    var2 = jnp.mean(jnp.square(z - mean2), axis=0, keepdims=True)
    z = (z - mean2) * lax.rsqrt(var2 + BN_EPS) * g2_ref[...] + b2_ref[...]
    hn = jnp.maximum(z, 0.0)
    h_out_ref[...] = hn
    pooled_ref[...] = jnp.sum(hn, axis=0, keepdims=True)


@jax.jit
def _dense_layer(h, partials, w1, g1, b1, w2, g2, b2):
    return pl.pallas_call(
        _dense_body,
        out_shape=(
            jax.ShapeDtypeStruct((N, D), jnp.float32),
            jax.ShapeDtypeStruct((1, D), jnp.float32),
        ),
    )(h, partials[0], partials[1], w1, g1.reshape(1, D), b1.reshape(1, D),
      w2, g2.reshape(1, D), b2.reshape(1, D))


def _pred_body(x_ref, p_ref, w_ref, b_ref, out_ref):
    pooled0 = jnp.sum(x_ref[...], axis=0, keepdims=True)
    pall = jnp.concatenate([pooled0, p_ref[...]], axis=0)
    per_layer = lax.dot_general(w_ref[...], pall,
                                (((2,), (1,)), ((0,), (0,))),
                                preferred_element_type=jnp.float32)
    out_ref[...] = (jnp.sum(per_layer, axis=0) +
                    jnp.sum(b_ref[...], axis=0)).reshape(1, OUT)


@jax.jit
def _final_pred(x, pooled, pred_w, pred_b):
    return pl.pallas_call(
        _pred_body,
        out_shape=jax.ShapeDtypeStruct((1, OUT), jnp.float32),
    )(x, pooled, pred_w, pred_b)


def kernel(x, edge_index, mlp_w1, mlp_bn_gamma, mlp_bn_beta, mlp_w2,
           bn_gamma, bn_beta, pred_w, pred_b):
    pad = E_PAD - E
    src = jnp.concatenate([edge_index[0], jnp.zeros((pad,), jnp.int32)])
    dst = jnp.concatenate(
        [edge_index[1], jnp.full((pad,), N, jnp.int32)]
    ).reshape(NW, E_PAD // NW // CHUNK, CHUNK)
    h = x
    pooled_list = []
    for i in range(L):
        partials = _seg_sum(h, src, dst)
        h, pooled = _dense_layer(h, partials, mlp_w1[i], mlp_bn_gamma[i],
                                 mlp_bn_beta[i], mlp_w2[i], bn_gamma[i],
                                 bn_beta[i])
        pooled_list.append(pooled)
    pooled_all = jnp.concatenate(pooled_list, axis=0)
    score = _final_pred(x, pooled_all, pred_w, pred_b)
    return score.reshape(OUT)


# speedup vs baseline: 2.7862x; 2.7862x over previous
import functools

import jax
import jax.numpy as jnp
from jax import lax
from jax.experimental import pallas as pl
from jax.experimental.pallas import tpu as pltpu
from jax.experimental.pallas import tpu_sc as plsc

N = 10000
E = 320000
D = 128
OUT = 16
L = 4
BN_EPS = 1e-5

NC = 2
NS = 16
NW = NC * NS

CHUNK = 80
N_CHUNKS = E // NW // CHUNK
N_AGG = N

RCHUNK = 16
NRC = N // RCHUNK
NRZ = NRC


def _seg_sum_body(src_hbm, dst_hbm, h_hbm, out_hbm,
                  src_v, dst_v, rows_a, rows_b, zero_v,
                  agg_sh, sem_a, sem_b):
    c = lax.axis_index("c")
    s = lax.axis_index("s")
    wid = c * NS + s

    cp_src = pltpu.async_copy(
        src_hbm.at[pl.ds(wid * N_CHUNKS * CHUNK, N_CHUNKS * CHUNK)],
        src_v, sem_a)
    cp_dst = pltpu.async_copy(dst_hbm.at[wid], dst_v, sem_b)

    @pl.loop(0, RCHUNK * D // 16)
    def _zero_vmem(i):
        r = i // (D // 16)
        k = i % (D // 16)
        zero_v[r, pl.ds(k * 16, 16)] = jnp.zeros((16,), jnp.float32)

    @pl.loop(0, (NRZ + NS - 1) // NS)
    def _zero_sh(j):
        ch = s + j * NS
        @pl.when(ch < NRZ)
        def _():
            pltpu.sync_copy(zero_v, agg_sh.at[pl.ds(ch * RCHUNK, RCHUNK)])

    cp_src.wait()
    cp_dst.wait()
    plsc.subcore_barrier()

    def _src_idx(j):
        return src_v.at[pl.ds(j * CHUNK, CHUNK)]

    pltpu.async_copy(h_hbm.at[_src_idx(0)], rows_a, sem_a)

    @pl.loop(0, N_CHUNKS)
    def _edges(j):
        even = j % 2 == 0

        @pl.when(even)
        def _():
            pltpu.make_async_copy(h_hbm.at[_src_idx(j)], rows_a, sem_a).wait()
            @pl.when(j + 1 < N_CHUNKS)
            def _():
                pltpu.async_copy(h_hbm.at[_src_idx(j + 1)], rows_b, sem_b)
            pltpu.sync_copy(rows_a, agg_sh.at[dst_v.at[j]], add=True)

        @pl.when(jnp.logical_not(even))
        def _():
            pltpu.make_async_copy(h_hbm.at[_src_idx(j)], rows_b, sem_b).wait()
            @pl.when(j + 1 < N_CHUNKS)
            def _():
                pltpu.async_copy(h_hbm.at[_src_idx(j + 1)], rows_a, sem_a)
            pltpu.sync_copy(rows_b, agg_sh.at[dst_v.at[j]], add=True)

    plsc.subcore_barrier()

    @pl.loop(0, (NRC + NS - 1) // NS)
    def _copy_out(j):
        ch = s + j * NS
        @pl.when(ch < NRC)
        def _():
            pltpu.sync_copy(agg_sh.at[pl.ds(ch * RCHUNK, RCHUNK)],
                            out_hbm.at[c].at[pl.ds(ch * RCHUNK, RCHUNK)])


@jax.jit
def _seg_sum(h, src, dst):
    mesh = plsc.VectorSubcoreMesh(core_axis_name="c", subcore_axis_name="s",
                                  num_cores=NC, num_subcores=NS)
    return pl.kernel(
        _seg_sum_body,
        out_type=jax.ShapeDtypeStruct((NC, N, D), jnp.float32),
        mesh=mesh,
        scratch_types=[
            pltpu.VMEM((N_CHUNKS * CHUNK,), jnp.int32),
            pltpu.VMEM((N_CHUNKS, CHUNK), jnp.int32),
            pltpu.VMEM((CHUNK, D), jnp.float32),
            pltpu.VMEM((CHUNK, D), jnp.float32),
            pltpu.VMEM((RCHUNK, D), jnp.float32),
            pltpu.VMEM_SHARED((N_AGG, D), jnp.float32),
            pltpu.SemaphoreType.DMA,
            pltpu.SemaphoreType.DMA,
        ],
    )(src, dst, h)


def _dense_body(h_ref, p0_ref, p1_ref, w1_ref, g1_ref, b1_ref,
                w2_ref, g2_ref, b2_ref, h_out_ref, pooled_ref):
    r = h_ref[...] + p0_ref[...] + p1_ref[...]
    y = lax.dot_general(r, w1_ref[...], (((1,), (1,)), ((), ())),
                        preferred_element_type=jnp.float32)
    mean = jnp.mean(y, axis=0, keepdims=True)
    var = jnp.mean(jnp.square(y - mean), axis=0, keepdims=True)
    y = (y - mean) * lax.rsqrt(var + BN_EPS) * g1_ref[...] + b1_ref[...]
    y = jnp.maximum(y, 0.0)
    z = lax.dot_general(y, w2_ref[...], (((1,), (1,)), ((), ())),
                        preferred_element_type=jnp.float32)
    mean2 = jnp.mean(z, axis=0, keepdims=True)
    var2 = jnp.mean(jnp.square(z - mean2), axis=0, keepdims=True)
    z = (z - mean2) * lax.rsqrt(var2 + BN_EPS) * g2_ref[...] + b2_ref[...]
    hn = jnp.maximum(z, 0.0)
    h_out_ref[...] = hn
    pooled_ref[...] = jnp.sum(hn, axis=0, keepdims=True)


@jax.jit
def _dense_layer(h, partials, w1, g1, b1, w2, g2, b2):
    return pl.pallas_call(
        _dense_body,
        out_shape=(
            jax.ShapeDtypeStruct((N, D), jnp.float32),
            jax.ShapeDtypeStruct((1, D), jnp.float32),
        ),
    )(h, partials[0], partials[1], w1, g1.reshape(1, D), b1.reshape(1, D),
      w2, g2.reshape(1, D), b2.reshape(1, D))


def _pred_body(x_ref, p_ref, w_ref, b_ref, out_ref):
    pooled0 = jnp.sum(x_ref[...], axis=0, keepdims=True)
    pall = jnp.concatenate([pooled0, p_ref[...]], axis=0)
    per_layer = lax.dot_general(w_ref[...], pall,
                                (((2,), (1,)), ((0,), (0,))),
                                preferred_element_type=jnp.float32)
    out_ref[...] = (jnp.sum(per_layer, axis=0) +
                    jnp.sum(b_ref[...], axis=0)).reshape(1, OUT)


@jax.jit
def _final_pred(x, pooled, pred_w, pred_b):
    return pl.pallas_call(
        _pred_body,
        out_shape=jax.ShapeDtypeStruct((1, OUT), jnp.float32),
    )(x, pooled, pred_w, pred_b)


def kernel(x, edge_index, mlp_w1, mlp_bn_gamma, mlp_bn_beta, mlp_w2,
           bn_gamma, bn_beta, pred_w, pred_b):
    src = edge_index[0]
    dst = edge_index[1].reshape(NW, N_CHUNKS, CHUNK)
    h = x
    pooled_list = []
    for i in range(L):
        partials = _seg_sum(h, src, dst)
        h, pooled = _dense_layer(h, partials, mlp_w1[i], mlp_bn_gamma[i],
                                 mlp_bn_beta[i], mlp_w2[i], bn_gamma[i],
                                 bn_beta[i])
        pooled_list.append(pooled)
    pooled_all = jnp.concatenate(pooled_list, axis=0)
    score = _final_pred(x, pooled_all, pred_w, pred_b)
    return score.reshape(OUT)


# speedup vs baseline: 4.1321x; 1.4831x over previous
import functools

import jax
import jax.numpy as jnp
from jax import lax
from jax.experimental import pallas as pl
from jax.experimental.pallas import tpu as pltpu
from jax.experimental.pallas import tpu_sc as plsc

N = 10000
E = 320000
D = 128
OUT = 16
L = 4
BN_EPS = 1e-5

NC = 2
NS = 16
NW = NC * NS

CHUNK = 80
N_CHUNKS = E // NW // CHUNK
N_AGG = N
NBUF = 4
NIDX = 8


def _seg_sum_body(src_hbm, dst_hbm, h_hbm, out_hbm,
                  src_v, dst_v, rows, agg_sh, sem_g, sem_s, sem_i):
    c = lax.axis_index("c")
    s = lax.axis_index("s")
    wid = c * NS + s
    ebase = wid * N_CHUNKS * CHUNK

    def idx_cp(ch, slot):
        a = pltpu.make_async_copy(src_hbm.at[pl.ds(ebase + ch * CHUNK, CHUNK)],
                                  src_v.at[slot], sem_i.at[slot])
        b = pltpu.make_async_copy(dst_hbm.at[pl.ds(ebase + ch * CHUNK, CHUNK)],
                                  dst_v.at[slot], sem_i.at[slot])
        return a, b

    def gather_cp(slot, b):
        return pltpu.make_async_copy(h_hbm.at[src_v.at[slot]], rows.at[b],
                                     sem_g.at[b])

    def scatter_cp(slot, b):
        return pltpu.make_async_copy(rows.at[b], agg_sh.at[dst_v.at[slot]],
                                     sem_s.at[b])

    @pl.loop(0, CHUNK * D // 16)
    def _zero_vmem(i):
        r = i // (D // 16)
        k = i % (D // 16)
        rows[0, r, pl.ds(k * 16, 16)] = jnp.zeros((16,), jnp.float32)

    @pl.loop(0, (N_CHUNKS + NS - 1) // NS)
    def _zero_sh(j):
        ch = s + j * NS
        @pl.when(ch < N_CHUNKS)
        def _():
            pltpu.sync_copy(rows.at[0], agg_sh.at[pl.ds(ch * CHUNK, CHUNK)])

    plsc.subcore_barrier()

    for ch in range(NBUF):
        a, b = idx_cp(ch, ch)
        a.start()
        b.start()
    for ch in range(2):
        a, b = idx_cp(ch, ch)
        a.wait()
        b.wait()
        gather_cp(ch, ch).start()

    @pl.loop(0, N_CHUNKS)
    def _edges(j):
        b = j % NBUF
        slot = j % NIDX
        gather_cp(slot, b).wait()
        scatter_cp(slot, b).start()

        @pl.when(j + 2 < N_CHUNKS)
        def _():
            j2 = j + 2
            bg = j2 % NBUF
            s2 = j2 % NIDX
            @pl.when(j >= 2)
            def _():
                scatter_cp((j - 2) % NIDX, bg).wait()
            ia, ib = idx_cp(j2, s2)
            ia.wait()
            ib.wait()
            gather_cp(s2, bg).start()

        @pl.when(j + NBUF < N_CHUNKS)
        def _():
            j4 = j + NBUF
            a, bb = idx_cp(j4, j4 % NIDX)
            a.start()
            bb.start()

    for ch in range(N_CHUNKS - NBUF, N_CHUNKS):
        scatter_cp(ch % NIDX, ch % NBUF).wait()

    plsc.subcore_barrier()

    @pl.loop(0, (N_CHUNKS + NS - 1) // NS)
    def _copy_out(j):
        ch = s + j * NS
        @pl.when(ch < N_CHUNKS)
        def _():
            pltpu.sync_copy(agg_sh.at[pl.ds(ch * CHUNK, CHUNK)],
                            out_hbm.at[c].at[pl.ds(ch * CHUNK, CHUNK)])


@jax.jit
def _seg_sum(h, src, dst):
    mesh = plsc.VectorSubcoreMesh(core_axis_name="c", subcore_axis_name="s",
                                  num_cores=NC, num_subcores=NS)
    return pl.kernel(
        _seg_sum_body,
        out_type=jax.ShapeDtypeStruct((NC, N, D), jnp.float32),
        mesh=mesh,
        scratch_types=[
            pltpu.VMEM((NIDX, CHUNK), jnp.int32),
            pltpu.VMEM((NIDX, CHUNK), jnp.int32),
            pltpu.VMEM((NBUF, CHUNK, D), jnp.float32),
            pltpu.VMEM_SHARED((N_AGG, D), jnp.float32),
            pltpu.SemaphoreType.DMA((NBUF,)),
            pltpu.SemaphoreType.DMA((NBUF,)),
            pltpu.SemaphoreType.DMA((NIDX,)),
        ],
    )(src, dst, h)


def _dense_body(h_ref, p0_ref, p1_ref, w1_ref, g1_ref, b1_ref,
                w2_ref, g2_ref, b2_ref, h_out_ref, pooled_ref):
    r = h_ref[...] + p0_ref[...] + p1_ref[...]
    y = lax.dot_general(r, w1_ref[...], (((1,), (1,)), ((), ())),
                        preferred_element_type=jnp.float32)
    mean = jnp.mean(y, axis=0, keepdims=True)
    var = jnp.mean(jnp.square(y - mean), axis=0, keepdims=True)
    y = (y - mean) * lax.rsqrt(var + BN_EPS) * g1_ref[...] + b1_ref[...]
    y = jnp.maximum(y, 0.0)
    z = lax.dot_general(y, w2_ref[...], (((1,), (1,)), ((), ())),
                        preferred_element_type=jnp.float32)
    mean2 = jnp.mean(z, axis=0, keepdims=True)
    var2 = jnp.mean(jnp.square(z - mean2), axis=0, keepdims=True)
    z = (z - mean2) * lax.rsqrt(var2 + BN_EPS) * g2_ref[...] + b2_ref[...]
    hn = jnp.maximum(z, 0.0)
    h_out_ref[...] = hn
    pooled_ref[...] = jnp.sum(hn, axis=0, keepdims=True)


@jax.jit
def _dense_layer(h, partials, w1, g1, b1, w2, g2, b2):
    return pl.pallas_call(
        _dense_body,
        out_shape=(
            jax.ShapeDtypeStruct((N, D), jnp.float32),
            jax.ShapeDtypeStruct((1, D), jnp.float32),
        ),
    )(h, partials[0], partials[1], w1, g1.reshape(1, D), b1.reshape(1, D),
      w2, g2.reshape(1, D), b2.reshape(1, D))


def _pred_body(x_ref, p_ref, w_ref, b_ref, out_ref):
    pooled0 = jnp.sum(x_ref[...], axis=0, keepdims=True)
    pall = jnp.concatenate([pooled0, p_ref[...]], axis=0)
    per_layer = lax.dot_general(w_ref[...], pall,
                                (((2,), (1,)), ((0,), (0,))),
                                preferred_element_type=jnp.float32)
    out_ref[...] = (jnp.sum(per_layer, axis=0) +
                    jnp.sum(b_ref[...], axis=0)).reshape(1, OUT)


@jax.jit
def _final_pred(x, pooled, pred_w, pred_b):
    return pl.pallas_call(
        _pred_body,
        out_shape=jax.ShapeDtypeStruct((1, OUT), jnp.float32),
    )(x, pooled, pred_w, pred_b)


def kernel(x, edge_index, mlp_w1, mlp_bn_gamma, mlp_bn_beta, mlp_w2,
           bn_gamma, bn_beta, pred_w, pred_b):
    src = edge_index[0]
    dst = edge_index[1]
    h = x
    pooled_list = []
    for i in range(L):
        partials = _seg_sum(h, src, dst)
        h, pooled = _dense_layer(h, partials, mlp_w1[i], mlp_bn_gamma[i],
                                 mlp_bn_beta[i], mlp_w2[i], bn_gamma[i],
                                 bn_beta[i])
        pooled_list.append(pooled)
    pooled_all = jnp.concatenate(pooled_list, axis=0)
    score = _final_pred(x, pooled_all, pred_w, pred_b)
    return score.reshape(OUT)
